# Initial kernel scaffold; baseline (speedup 1.0000x reference)
import jax
import jax.numpy as jnp
from jax.experimental import pallas as pl


def kernel(x_start, t, noise, sqrt_alphas_cumprod, sqrt_one_minus_alphas_cumprod):
    raise NotImplementedError("write your pallas kernel here")


# speedup vs baseline: 2.1665x; 2.1665x over previous
import jax
import jax.numpy as jnp
from jax.experimental import pallas as pl

_NUM_SCALES = 1000
_TPAD = 1024


def _body(t_ref, tab_ref, x_ref, n_ref, o_ref):
    tcol = t_ref[0]
    br = tcol.shape[0]
    iot = jax.lax.broadcasted_iota(jnp.int32, (br, _TPAD), 1)
    oneh = (tcol == iot).astype(jnp.float32)
    coef = jnp.dot(oneh, tab_ref[...], preferred_element_type=jnp.float32,
                   precision=jax.lax.Precision.HIGHEST)
    a = coef[:, 0:1]
    b = coef[:, 1:2]
    o_ref[...] = a * x_ref[...] + b * n_ref[...]


def kernel(x_start, t, noise, sqrt_alphas_cumprod, sqrt_one_minus_alphas_cumprod):
    batch, dim = x_start.shape
    br = 256
    nb = batch // br
    t3 = t.astype(jnp.int32).reshape(nb, br, 1)
    tab = jnp.zeros((_TPAD, 2), jnp.float32)
    tab = tab.at[:_NUM_SCALES, 0].set(sqrt_alphas_cumprod)
    tab = tab.at[:_NUM_SCALES, 1].set(sqrt_one_minus_alphas_cumprod)
    return pl.pallas_call(
        _body,
        grid=(nb,),
        in_specs=[
            pl.BlockSpec((1, br, 1), lambda i: (i, 0, 0)),
            pl.BlockSpec((_TPAD, 2), lambda i: (0, 0)),
            pl.BlockSpec((br, dim), lambda i: (i, 0)),
            pl.BlockSpec((br, dim), lambda i: (i, 0)),
        ],
        out_specs=pl.BlockSpec((br, dim), lambda i: (i, 0)),
        out_shape=jax.ShapeDtypeStruct((batch, dim), jnp.float32),
    )(t3, tab, x_start, noise)


# speedup vs baseline: 2.3286x; 1.0748x over previous
import jax
import jax.numpy as jnp
from jax.experimental import pallas as pl

_NUM_SCALES = 1000
_TPAD = 1024


def _body(t_ref, tab_ref, ones_ref, x_ref, n_ref, o_ref):
    tcol = t_ref[0]
    br = tcol.shape[0]
    hi = tcol >> 7
    lo = tcol & 127
    oh_hi = (hi == jax.lax.broadcasted_iota(jnp.int32, (br, 8), 1)).astype(jnp.float32)
    oh_lo = lo == jax.lax.broadcasted_iota(jnp.int32, (br, 128), 1)
    rows = jnp.dot(oh_hi, tab_ref[...], preferred_element_type=jnp.float32,
                   precision=jax.lax.Precision.HIGHEST)
    masked = jnp.where(jnp.concatenate([oh_lo, oh_lo], axis=1), rows, 0.0)
    coef = jnp.dot(masked, ones_ref[...], preferred_element_type=jnp.float32,
                   precision=jax.lax.Precision.HIGHEST)
    a = coef[:, 0:1]
    b = coef[:, 1:2]
    o_ref[...] = a * x_ref[...] + b * n_ref[...]


def kernel(x_start, t, noise, sqrt_alphas_cumprod, sqrt_one_minus_alphas_cumprod):
    batch, dim = x_start.shape
    br = 256
    nb = batch // br
    t3 = t.astype(jnp.int32).reshape(nb, br, 1)
    taba = jnp.zeros((_TPAD,), jnp.float32).at[:_NUM_SCALES].set(
        sqrt_alphas_cumprod).reshape(8, 128)
    tabb = jnp.zeros((_TPAD,), jnp.float32).at[:_NUM_SCALES].set(
        sqrt_one_minus_alphas_cumprod).reshape(8, 128)
    tab = jnp.concatenate([taba, tabb], axis=1)
    lane = jnp.arange(256)
    ones = jnp.stack([(lane < 128).astype(jnp.float32),
                      (lane >= 128).astype(jnp.float32)], axis=1)
    return pl.pallas_call(
        _body,
        grid=(nb,),
        in_specs=[
            pl.BlockSpec((1, br, 1), lambda i: (i, 0, 0)),
            pl.BlockSpec((8, 256), lambda i: (0, 0)),
            pl.BlockSpec((256, 2), lambda i: (0, 0)),
            pl.BlockSpec((br, dim), lambda i: (i, 0)),
            pl.BlockSpec((br, dim), lambda i: (i, 0)),
        ],
        out_specs=pl.BlockSpec((br, dim), lambda i: (i, 0)),
        out_shape=jax.ShapeDtypeStruct((batch, dim), jnp.float32),
    )(t3, tab, ones, x_start, noise)


# speedup vs baseline: 2.7256x; 1.1705x over previous
import jax
import jax.numpy as jnp
from jax.experimental import pallas as pl

_NUM_SCALES = 1000
_TPAD = 1024


def _body(t_ref, tab_ref, ones_ref, x_ref, n_ref, o_ref):
    tcol = t_ref[0]
    br = tcol.shape[0]
    hi = tcol >> 7
    lo = tcol & 127
    oh_hi = (hi == jax.lax.broadcasted_iota(jnp.int32, (br, 8), 1)).astype(jnp.float32)
    oh_lo = lo == jax.lax.broadcasted_iota(jnp.int32, (br, 128), 1)
    rows = jnp.dot(oh_hi, tab_ref[...], preferred_element_type=jnp.float32,
                   precision=jax.lax.Precision.HIGHEST)
    masked = jnp.where(jnp.concatenate([oh_lo, oh_lo], axis=1), rows, 0.0)
    coef = jnp.dot(masked, ones_ref[...], preferred_element_type=jnp.float32,
                   precision=jax.lax.Precision.HIGHEST)
    a = coef[:, 0:1]
    b = coef[:, 1:2]
    o_ref[...] = a * x_ref[...] + b * n_ref[...]


def kernel(x_start, t, noise, sqrt_alphas_cumprod, sqrt_one_minus_alphas_cumprod):
    batch, dim = x_start.shape
    br = 512
    nb = batch // br
    t3 = t.astype(jnp.int32).reshape(nb, br, 1)
    taba = jnp.zeros((_TPAD,), jnp.float32).at[:_NUM_SCALES].set(
        sqrt_alphas_cumprod).reshape(8, 128)
    tabb = jnp.zeros((_TPAD,), jnp.float32).at[:_NUM_SCALES].set(
        sqrt_one_minus_alphas_cumprod).reshape(8, 128)
    tab = jnp.concatenate([taba, tabb], axis=1)
    lane = jnp.arange(256)
    ones = jnp.stack([(lane < 128).astype(jnp.float32),
                      (lane >= 128).astype(jnp.float32)], axis=1)
    return pl.pallas_call(
        _body,
        grid=(nb,),
        in_specs=[
            pl.BlockSpec((1, br, 1), lambda i: (i, 0, 0)),
            pl.BlockSpec((8, 256), lambda i: (0, 0)),
            pl.BlockSpec((256, 2), lambda i: (0, 0)),
            pl.BlockSpec((br, dim), lambda i: (i, 0)),
            pl.BlockSpec((br, dim), lambda i: (i, 0)),
        ],
        out_specs=pl.BlockSpec((br, dim), lambda i: (i, 0)),
        out_shape=jax.ShapeDtypeStruct((batch, dim), jnp.float32),
    )(t3, tab, ones, x_start, noise)


# speedup vs baseline: 2.9596x; 1.0858x over previous
import jax
import jax.numpy as jnp
from jax.experimental import pallas as pl

_NUM_SCALES = 1000
_TPAD = 1024


def _body(t_ref, tab_ref, ones_ref, x_ref, n_ref, o_ref):
    tcol = t_ref[0]
    br = tcol.shape[0]
    hi = tcol >> 7
    lo = tcol & 127
    oh_hi = (hi == jax.lax.broadcasted_iota(jnp.int32, (br, 8), 1)).astype(jnp.float32)
    oh_lo = lo == jax.lax.broadcasted_iota(jnp.int32, (br, 128), 1)
    rows = jnp.dot(oh_hi, tab_ref[...], preferred_element_type=jnp.float32,
                   precision=jax.lax.Precision.HIGHEST)
    masked = jnp.where(jnp.concatenate([oh_lo, oh_lo], axis=1), rows, 0.0)
    coef = jnp.dot(masked, ones_ref[...], preferred_element_type=jnp.float32,
                   precision=jax.lax.Precision.HIGHEST)
    a = coef[:, 0:1]
    b = coef[:, 1:2]
    o_ref[...] = a * x_ref[...] + b * n_ref[...]


def kernel(x_start, t, noise, sqrt_alphas_cumprod, sqrt_one_minus_alphas_cumprod):
    batch, dim = x_start.shape
    br = 1024
    nb = batch // br
    t3 = t.astype(jnp.int32).reshape(nb, br, 1)
    taba = jnp.zeros((_TPAD,), jnp.float32).at[:_NUM_SCALES].set(
        sqrt_alphas_cumprod).reshape(8, 128)
    tabb = jnp.zeros((_TPAD,), jnp.float32).at[:_NUM_SCALES].set(
        sqrt_one_minus_alphas_cumprod).reshape(8, 128)
    tab = jnp.concatenate([taba, tabb], axis=1)
    lane = jnp.arange(256)
    ones = jnp.stack([(lane < 128).astype(jnp.float32),
                      (lane >= 128).astype(jnp.float32)], axis=1)
    return pl.pallas_call(
        _body,
        grid=(nb,),
        in_specs=[
            pl.BlockSpec((1, br, 1), lambda i: (i, 0, 0)),
            pl.BlockSpec((8, 256), lambda i: (0, 0)),
            pl.BlockSpec((256, 2), lambda i: (0, 0)),
            pl.BlockSpec((br, dim), lambda i: (i, 0)),
            pl.BlockSpec((br, dim), lambda i: (i, 0)),
        ],
        out_specs=pl.BlockSpec((br, dim), lambda i: (i, 0)),
        out_shape=jax.ShapeDtypeStruct((batch, dim), jnp.float32),
    )(t3, tab, ones, x_start, noise)


# speedup vs baseline: 3.0560x; 1.0326x over previous
import jax
import jax.numpy as jnp
from jax.experimental import pallas as pl

_NUM_SCALES = 1000
_TPAD = 1024


def _body(t_ref, tab_ref, ones_ref, x_ref, n_ref, o_ref):
    tcol = t_ref[0]
    br = tcol.shape[0]
    hi = tcol >> 7
    lo = tcol & 127
    oh_hi = (hi == jax.lax.broadcasted_iota(jnp.int32, (br, 8), 1)).astype(jnp.float32)
    oh_lo = lo == jax.lax.broadcasted_iota(jnp.int32, (br, 128), 1)
    rows = jnp.dot(oh_hi, tab_ref[...], preferred_element_type=jnp.float32,
                   precision=jax.lax.Precision.HIGHEST)
    masked = jnp.where(jnp.concatenate([oh_lo, oh_lo], axis=1), rows, 0.0)
    coef = jnp.dot(masked, ones_ref[...], preferred_element_type=jnp.float32,
                   precision=jax.lax.Precision.HIGHEST)
    a = coef[:, 0:1]
    b = coef[:, 1:2]
    o_ref[...] = a * x_ref[...] + b * n_ref[...]


def kernel(x_start, t, noise, sqrt_alphas_cumprod, sqrt_one_minus_alphas_cumprod):
    batch, dim = x_start.shape
    br = 2048
    nb = batch // br
    t3 = t.astype(jnp.int32).reshape(nb, br, 1)
    taba = jnp.zeros((_TPAD,), jnp.float32).at[:_NUM_SCALES].set(
        sqrt_alphas_cumprod).reshape(8, 128)
    tabb = jnp.zeros((_TPAD,), jnp.float32).at[:_NUM_SCALES].set(
        sqrt_one_minus_alphas_cumprod).reshape(8, 128)
    tab = jnp.concatenate([taba, tabb], axis=1)
    lane = jnp.arange(256)
    ones = jnp.stack([(lane < 128).astype(jnp.float32),
                      (lane >= 128).astype(jnp.float32)], axis=1)
    return pl.pallas_call(
        _body,
        grid=(nb,),
        in_specs=[
            pl.BlockSpec((1, br, 1), lambda i: (i, 0, 0)),
            pl.BlockSpec((8, 256), lambda i: (0, 0)),
            pl.BlockSpec((256, 2), lambda i: (0, 0)),
            pl.BlockSpec((br, dim), lambda i: (i, 0)),
            pl.BlockSpec((br, dim), lambda i: (i, 0)),
        ],
        out_specs=pl.BlockSpec((br, dim), lambda i: (i, 0)),
        out_shape=jax.ShapeDtypeStruct((batch, dim), jnp.float32),
    )(t3, tab, ones, x_start, noise)


# speedup vs baseline: 6.4184x; 2.1003x over previous
import jax
import jax.numpy as jnp
from jax.experimental import pallas as pl

_NUM_SCALES = 1000
_TPAD = 1024


def _body(t_ref, tab_ref, x_ref, n_ref, o_ref):
    tcol = t_ref[0]
    br = tcol.shape[0]
    hi = tcol >> 7
    lo = tcol & 127
    oh_hi = (hi == jax.lax.broadcasted_iota(jnp.int32, (br, 8), 1)).astype(jnp.float32)
    oh_lo = lo == jax.lax.broadcasted_iota(jnp.int32, (br, 128), 1)
    rows = jnp.dot(oh_hi, tab_ref[...], preferred_element_type=jnp.float32,
                   precision=jax.lax.Precision.HIGHEST)
    a = jnp.sum(jnp.where(oh_lo, rows[:, :128], 0.0), axis=1, keepdims=True)
    b = jnp.sum(jnp.where(oh_lo, rows[:, 128:], 0.0), axis=1, keepdims=True)
    o_ref[...] = a * x_ref[...] + b * n_ref[...]


def kernel(x_start, t, noise, sqrt_alphas_cumprod, sqrt_one_minus_alphas_cumprod):
    batch, dim = x_start.shape
    br = 2048
    nb = batch // br
    t3 = t.astype(jnp.int32).reshape(nb, br, 1)
    taba = jnp.zeros((_TPAD,), jnp.float32).at[:_NUM_SCALES].set(
        sqrt_alphas_cumprod).reshape(8, 128)
    tabb = jnp.zeros((_TPAD,), jnp.float32).at[:_NUM_SCALES].set(
        sqrt_one_minus_alphas_cumprod).reshape(8, 128)
    tab = jnp.concatenate([taba, tabb], axis=1)
    return pl.pallas_call(
        _body,
        grid=(nb,),
        in_specs=[
            pl.BlockSpec((1, br, 1), lambda i: (i, 0, 0)),
            pl.BlockSpec((8, 256), lambda i: (0, 0)),
            pl.BlockSpec((br, dim), lambda i: (i, 0)),
            pl.BlockSpec((br, dim), lambda i: (i, 0)),
        ],
        out_specs=pl.BlockSpec((br, dim), lambda i: (i, 0)),
        out_shape=jax.ShapeDtypeStruct((batch, dim), jnp.float32),
    )(t3, tab, x_start, noise)


# speedup vs baseline: 10.0532x; 1.5663x over previous
import jax
import jax.numpy as jnp
from jax.experimental import pallas as pl

_NUM_SCALES = 1000
_TPAD = 1024


def _body(t_ref, tab_ref, x_ref, n_ref, o_ref):
    tcol = t_ref[...][:, None]
    br = tcol.shape[0]
    hi = tcol >> 7
    lo = tcol & 127
    oh_hi = (hi == jax.lax.broadcasted_iota(jnp.int32, (br, 8), 1)).astype(jnp.float32)
    oh_lo = lo == jax.lax.broadcasted_iota(jnp.int32, (br, 128), 1)
    rows = jnp.dot(oh_hi, tab_ref[...], preferred_element_type=jnp.float32,
                   precision=jax.lax.Precision.HIGHEST)
    a = jnp.sum(jnp.where(oh_lo, rows[:, :128], 0.0), axis=1, keepdims=True)
    b = jnp.sum(jnp.where(oh_lo, rows[:, 128:], 0.0), axis=1, keepdims=True)
    o_ref[...] = a * x_ref[...] + b * n_ref[...]


def kernel(x_start, t, noise, sqrt_alphas_cumprod, sqrt_one_minus_alphas_cumprod):
    batch, dim = x_start.shape
    br = 2048
    nb = batch // br
    t1 = t.astype(jnp.int32)
    taba = jnp.zeros((_TPAD,), jnp.float32).at[:_NUM_SCALES].set(
        sqrt_alphas_cumprod).reshape(8, 128)
    tabb = jnp.zeros((_TPAD,), jnp.float32).at[:_NUM_SCALES].set(
        sqrt_one_minus_alphas_cumprod).reshape(8, 128)
    tab = jnp.concatenate([taba, tabb], axis=1)
    return pl.pallas_call(
        _body,
        grid=(nb,),
        in_specs=[
            pl.BlockSpec((br,), lambda i: (i,)),
            pl.BlockSpec((8, 256), lambda i: (0, 0)),
            pl.BlockSpec((br, dim), lambda i: (i, 0)),
            pl.BlockSpec((br, dim), lambda i: (i, 0)),
        ],
        out_specs=pl.BlockSpec((br, dim), lambda i: (i, 0)),
        out_shape=jax.ShapeDtypeStruct((batch, dim), jnp.float32),
    )(t1, tab, x_start, noise)
